# baseline (speedup 1.0000x reference)
import functools

import jax
import jax.numpy as jnp
from jax import lax
from jax.experimental import pallas as pl
from jax.experimental.pallas import tpu as pltpu
from jax.experimental.pallas import tpu_sc as plsc

VOCAB = 1000000
DIM = 64
BATCH = 16384
X_MAX = 100.0
ALPHA = 0.75

NC = 2
NS = 16
NW = NC * NS
L = 16
BPW = BATCH // NW

RANGE = 31232
MAIN_END = 999936
TAIL_BI = MAIN_END // 128 - 31 * (RANGE // 128)
WLCAP = BATCH + L


def _scan_extract(wl, cnt, src, ring, semw, o_ref, tmp, dvbig, mtot, bi):
    lanes = lax.iota(jnp.int32, L)
    nvregs = (cnt + L - 1) >> 4

    def scanv(v, mtot):
        vals = wl[pl.ds(v * L, L)]
        mm = ((vals & 32767) >> 7) == bi
        mm = mm & ((lanes + v * L) < cnt)
        plsc.store_compressed(tmp.at[pl.ds(0, L)], vals, mask=mm)
        nv = plsc.all_reduce_population_count(mm)[0]

        def handle(e, mtot):
            packed = tmp[pl.ds(e, L)][0]
            kpair = packed >> 15
            cloc = (packed & 32767) - bi * 128
            colv = jnp.full((L,), 0, jnp.int32) + cloc
            slot = mtot & 15
            for r in range(DIM // L):
                ring[slot, pl.ds(r * L, L)] = plsc.load_gather(
                    src, [lanes + r * L, colv])
            pltpu.async_copy(
                ring.at[slot], o_ref.at[pl.ds(kpair * DIM, DIM)], semw)

            @pl.when(slot == 15)
            def _():
                pltpu.make_async_copy(
                    o_ref.at[pl.ds(0, 16 * DIM)], dvbig, semw).wait()

            return mtot + 1

        return lax.fori_loop(0, nv, handle, mtot)

    return lax.fori_loop(0, nvregs, scanv, mtot)


def _k1_body(pt_hbm, ptt_hbm, i_hbm, j_hbm, wtl_hbm, wttl_hbm,
             owi_hbm, owj_hbm,
             ivall, wli, wlj,
             bw0, bw1, bw2, bw3, bt0, bt1, bt2, bt3, wtv, wttv,
             ring1, ring2, tmp, dvbig, dv,
             semb0, semb1, semb2, semb3, sem1, sem2):
    wid = lax.axis_index("s") * NC + lax.axis_index("c")
    lo = wid * RANGE
    is_last = wid == NW - 1
    nb_main = jnp.where(is_last, TAIL_BI, RANGE // 128)
    lanes = lax.iota(jnp.int32, L)

    pltpu.sync_copy(wtl_hbm, wtv)
    pltpu.sync_copy(wttl_hbm, wttv)

    ICH = 4096

    def build_wl(src_hbm, wl):
        hi = jnp.where(is_last, VOCAB, lo + RANGE)

        def chunk(ci, cnt):
            pltpu.sync_copy(src_hbm.at[pl.ds(ci * ICH, ICH)], ivall)

            def scan(v, cnt):
                vals = ivall[pl.ds(v * L, L)]
                m = (vals >= lo) & (vals < hi)
                packed = ((lanes + ci * ICH + v * L) << 15) | (vals - lo)
                plsc.store_compressed(wl.at[pl.ds(cnt, L)], packed, mask=m)
                return cnt + plsc.all_reduce_population_count(m)[0]

            return lax.fori_loop(0, ICH // L, scan, cnt)

        return lax.fori_loop(0, BATCH // ICH, chunk, 0)

    cnt_i = build_wl(i_hbm, wli)
    cnt_j = build_wl(j_hbm, wlj)

    bufs = ((bw0, bt0, semb0), (bw1, bt1, semb1),
            (bw2, bt2, semb2), (bw3, bt3, semb3))

    def fire(bi, bw, bt, semb):
        col = lo + bi * 128
        pltpu.async_copy(pt_hbm.at[:, pl.ds(col, 128)], bw, semb)
        pltpu.async_copy(ptt_hbm.at[:, pl.ds(col, 128)], bt, semb)

    def drain_blk(bw, bt, semb):
        pltpu.make_async_copy(pt_hbm.at[:, pl.ds(0, 128)], bw, semb).wait()
        pltpu.make_async_copy(ptt_hbm.at[:, pl.ds(0, 128)], bt, semb).wait()

    for pre in (0, 1, 2):
        fire(pre, *bufs[pre])

    def block4(bi4, carry):
        m1, m2 = carry
        for b in (0, 1, 2, 3):
            bi = bi4 * 4 + b
            bw, bt, semb = bufs[b]
            nbw, nbt, nsemb = bufs[(b + 3) & 3]
            drain_blk(bw, bt, semb)

            @pl.when(bi + 3 < nb_main)
            def _():
                fire(bi + 3, nbw, nbt, nsemb)

            m1 = _scan_extract(wli, cnt_i, bw, ring1, sem1, owi_hbm,
                               tmp, dvbig, m1, bi)
            m2 = _scan_extract(wlj, cnt_j, bt, ring2, sem2, owj_hbm,
                               tmp, dvbig, m2, bi)
        return (m1, m2)

    m1, m2 = lax.fori_loop(0, nb_main >> 2, block4, (0, 0))

    m1 = _scan_extract(wli, cnt_i, wtv, ring1, sem1, owi_hbm,
                       tmp, dvbig, m1, TAIL_BI)
    m2 = _scan_extract(wlj, cnt_j, wttv, ring2, sem2, owj_hbm,
                       tmp, dvbig, m2, TAIL_BI)

    def drain_rest(m, semw, o_ref):
        def one(e, _):
            pltpu.make_async_copy(o_ref.at[pl.ds(0, DIM)], dv, semw).wait()
            return 0
        lax.fori_loop(0, m & 15, one, 0)

    drain_rest(m1, sem1, owi_hbm)
    drain_rest(m2, sem2, owj_hbm)


_k1 = functools.partial(
    pl.kernel,
    out_type=(
        jax.ShapeDtypeStruct((BATCH * DIM,), jnp.float32),
        jax.ShapeDtypeStruct((BATCH * DIM,), jnp.float32),
    ),
    mesh=plsc.VectorSubcoreMesh(core_axis_name="c", subcore_axis_name="s"),
    compiler_params=pltpu.CompilerParams(
        needs_layout_passes=False, use_tc_tiling_on_sc=True
    ),
    scratch_types=[
        pltpu.VMEM((4096,), jnp.int32),
        pltpu.VMEM((WLCAP,), jnp.int32),
        pltpu.VMEM((WLCAP,), jnp.int32),
        pltpu.VMEM((DIM, 128), jnp.float32),
        pltpu.VMEM((DIM, 128), jnp.float32),
        pltpu.VMEM((DIM, 128), jnp.float32),
        pltpu.VMEM((DIM, 128), jnp.float32),
        pltpu.VMEM((DIM, 128), jnp.float32),
        pltpu.VMEM((DIM, 128), jnp.float32),
        pltpu.VMEM((DIM, 128), jnp.float32),
        pltpu.VMEM((DIM, 128), jnp.float32),
        pltpu.VMEM((DIM, 128), jnp.float32),
        pltpu.VMEM((DIM, 128), jnp.float32),
        pltpu.VMEM((16, DIM), jnp.float32),
        pltpu.VMEM((16, DIM), jnp.float32),
        pltpu.VMEM((2 * L,), jnp.int32),
        pltpu.VMEM((16 * DIM,), jnp.float32),
        pltpu.VMEM((DIM,), jnp.float32),
        pltpu.SemaphoreType.DMA,
        pltpu.SemaphoreType.DMA,
        pltpu.SemaphoreType.DMA,
        pltpu.SemaphoreType.DMA,
        pltpu.SemaphoreType.DMA,
        pltpu.SemaphoreType.DMA,
    ],
)(_k1_body)


def _k2_body(wif_hbm, wjf_hbm, i_hbm, j_hbm, b_hbm, bt_hbm, out_hbm,
             iv, jv, biv, bjv, wiv, wjv, pv, sem):
    wid = lax.axis_index("s") * NC + lax.axis_index("c")
    base = wid * BPW

    pltpu.sync_copy(i_hbm.at[pl.ds(base, BPW)], iv.at[pl.ds(0, BPW)])
    pltpu.sync_copy(j_hbm.at[pl.ds(base, BPW)], jv.at[pl.ds(0, BPW)])
    zeros16i = jnp.zeros((L,), jnp.int32)
    iv[pl.ds(BPW, L)] = zeros16i
    jv[pl.ds(BPW, L)] = zeros16i

    cb = pltpu.async_copy(b_hbm.at[iv], biv, sem)
    cb.wait()
    cbt = pltpu.async_copy(bt_hbm.at[jv], bjv, sem)
    cbt.wait()

    pltpu.sync_copy(wif_hbm.at[pl.ds(base * DIM, BPW * DIM)], wiv)
    pltpu.sync_copy(wjf_hbm.at[pl.ds(base * DIM, BPW * DIM)], wjv)

    lane = lax.iota(jnp.int32, L)

    def pair(p, _):
        acc = jnp.zeros((L,), jnp.float32)
        for r in range(DIM // L):
            a = wiv[pl.ds(p * DIM + r * L, L)]
            c = wjv[pl.ds(p * DIM + r * L, L)]
            acc = acc + a * c
        bsum = biv[pl.ds(p, L)][0] + bjv[pl.ds(p, L)][0]
        acc = jnp.where(lane == 0, acc + bsum, acc)
        pv[p >> 3, pl.ds((p & 7) * L, L)] = acc
        return 0

    lax.fori_loop(0, BPW, pair, 0)
    pltpu.sync_copy(pv, out_hbm.at[pl.ds(wid * (BPW // 8), BPW // 8)])


_k2 = functools.partial(
    pl.kernel,
    out_type=jax.ShapeDtypeStruct((BATCH // 8, 128), jnp.float32),
    mesh=plsc.VectorSubcoreMesh(core_axis_name="c", subcore_axis_name="s"),
    compiler_params=pltpu.CompilerParams(
        needs_layout_passes=False, use_tc_tiling_on_sc=False
    ),
    scratch_types=[
        pltpu.VMEM((BPW + L,), jnp.int32),
        pltpu.VMEM((BPW + L,), jnp.int32),
        pltpu.VMEM((BPW + L,), jnp.float32),
        pltpu.VMEM((BPW + L,), jnp.float32),
        pltpu.VMEM((BPW * DIM,), jnp.float32),
        pltpu.VMEM((BPW * DIM,), jnp.float32),
        pltpu.VMEM((BPW // 8, 128), jnp.float32),
        pltpu.SemaphoreType.DMA,
    ],
)(_k2_body)


def _tc_loss_body(p_ref, x_ref, o_ref):
    p = p_ref[...]
    xr = x_ref[...]
    cc = lax.broadcasted_iota(jnp.int32, (128, 8), 0)
    mm = lax.broadcasted_iota(jnp.int32, (128, 8), 1)
    blk = jnp.where(cc // 16 == mm, 1.0, 0.0).astype(jnp.float32)
    s = jax.lax.dot_general(
        p, blk, (((1,), (0,)), ((), ())),
        precision=jax.lax.Precision.HIGHEST,
        preferred_element_type=jnp.float32,
    )
    diff = s - jnp.log(xr)
    w = jnp.clip((xr * (1.0 / X_MAX)) ** ALPHA, 0.0, 1.0)
    o_ref[0, 0] = jnp.sum(w * diff * diff) * (1.0 / BATCH)


_tc_loss = pl.pallas_call(
    _tc_loss_body,
    out_shape=jax.ShapeDtypeStruct((1, 1), jnp.float32),
    out_specs=pl.BlockSpec(memory_space=pltpu.MemorySpace.SMEM),
)


def kernel(i, j, x, W, W_tilde, b, b_tilde):
    i = i.astype(jnp.int32)
    j = j.astype(jnp.int32)
    wtail = jnp.pad(W[MAIN_END:].T, ((0, 0), (0, 128 - (VOCAB - MAIN_END))))
    wttail = jnp.pad(W_tilde[MAIN_END:].T,
                     ((0, 0), (0, 128 - (VOCAB - MAIN_END))))
    wi_flat, wj_flat = _k1(W.T, W_tilde.T, i, j, wtail, wttail)
    partials = _k2(wi_flat, wj_flat, i, j, b, b_tilde)
    out = _tc_loss(partials, x.reshape(BATCH // 8, 8))
    return out[0, 0]


# speedup vs baseline: 1.5725x; 1.5725x over previous
import functools

import jax
import jax.numpy as jnp
from jax import lax
from jax.experimental import pallas as pl
from jax.experimental.pallas import tpu as pltpu
from jax.experimental.pallas import tpu_sc as plsc

VOCAB = 1000000
DIM = 64
BATCH = 16384
X_MAX = 100.0
ALPHA = 0.75

NC = 2
NS = 16
NW = NC * NS
L = 16
BPW = BATCH // NW

RANGE = 31232
MAIN_END = 999936
SLAB_W = 256
TAIL_BI = (MAIN_END - 31 * RANGE) // SLAB_W
WLCAP = BATCH + L


def _scan_extract(wl, cnt, src, ring, semw, o_ref, tmp, dvbig, mtot, bi):
    lanes = lax.iota(jnp.int32, L)
    nvregs = (cnt + L - 1) >> 4

    def scanv(v, mtot):
        vals = wl[pl.ds(v * L, L)]
        mm = ((vals & 32767) >> 8) == bi
        mm = mm & ((lanes + v * L) < cnt)
        plsc.store_compressed(tmp.at[pl.ds(0, L)], vals, mask=mm)
        nv = plsc.all_reduce_population_count(mm)[0]

        def handle(e, mtot):
            packed = tmp[pl.ds(e, L)][0]
            kpair = packed >> 15
            cloc = (packed & 32767) - bi * SLAB_W
            colv = jnp.full((L,), 0, jnp.int32) + cloc
            slot = mtot & 15
            for r in range(DIM // L):
                ring[slot, pl.ds(r * L, L)] = plsc.load_gather(
                    src, [lanes + r * L, colv])
            pltpu.async_copy(
                ring.at[slot], o_ref.at[pl.ds(kpair * DIM, DIM)], semw)

            @pl.when(slot == 15)
            def _():
                pltpu.make_async_copy(
                    o_ref.at[pl.ds(0, 16 * DIM)], dvbig, semw).wait()

            return mtot + 1

        return lax.fori_loop(0, nv, handle, mtot)

    return lax.fori_loop(0, nvregs, scanv, mtot)


def _k1_body(pt_hbm, ptt_hbm, i_hbm, j_hbm, wtl_hbm, wttl_hbm,
             owi_hbm, owj_hbm,
             ivall, wli, wlj,
             bw0, bw1, bt0, bt1, wtv, wttv,
             ring1, ring2, tmp, dvbig, dv,
             semb0, semb1, sem1, sem2):
    wid = lax.axis_index("s") * NC + lax.axis_index("c")
    lo = wid * RANGE
    is_last = wid == NW - 1
    nb_main = jnp.where(is_last, TAIL_BI, RANGE // SLAB_W)
    lanes = lax.iota(jnp.int32, L)

    pltpu.sync_copy(wtl_hbm, wtv)
    pltpu.sync_copy(wttl_hbm, wttv)

    ICH = 4096

    def build_wl(src_hbm, wl):
        hi = jnp.where(is_last, VOCAB, lo + RANGE)

        def chunk(ci, cnt):
            pltpu.sync_copy(src_hbm.at[pl.ds(ci * ICH, ICH)], ivall)

            def scan(v, cnt):
                vals = ivall[pl.ds(v * L, L)]
                m = (vals >= lo) & (vals < hi)
                packed = ((lanes + ci * ICH + v * L) << 15) | (vals - lo)
                plsc.store_compressed(wl.at[pl.ds(cnt, L)], packed, mask=m)
                return cnt + plsc.all_reduce_population_count(m)[0]

            return lax.fori_loop(0, ICH // L, scan, cnt)

        return lax.fori_loop(0, BATCH // ICH, chunk, 0)

    cnt_i = build_wl(i_hbm, wli)
    cnt_j = build_wl(j_hbm, wlj)

    bufs = ((bw0, bt0, semb0), (bw1, bt1, semb1))

    def fire(bi, bw, bt, semb):
        col = lo + bi * SLAB_W
        pltpu.async_copy(pt_hbm.at[:, pl.ds(col, SLAB_W)], bw, semb)
        pltpu.async_copy(ptt_hbm.at[:, pl.ds(col, SLAB_W)], bt, semb)

    def drain_blk(bw, bt, semb):
        pltpu.make_async_copy(pt_hbm.at[:, pl.ds(0, SLAB_W)], bw, semb).wait()
        pltpu.make_async_copy(ptt_hbm.at[:, pl.ds(0, SLAB_W)], bt, semb).wait()

    fire(0, *bufs[0])

    def block2(bi2, carry):
        m1, m2 = carry
        for b in (0, 1):
            bi = bi2 * 2 + b
            bw, bt, semb = bufs[b]
            nbw, nbt, nsemb = bufs[1 - b]
            drain_blk(bw, bt, semb)

            @pl.when(bi + 1 < nb_main)
            def _():
                fire(bi + 1, nbw, nbt, nsemb)

            m1 = _scan_extract(wli, cnt_i, bw, ring1, sem1, owi_hbm,
                               tmp, dvbig, m1, bi)
            m2 = _scan_extract(wlj, cnt_j, bt, ring2, sem2, owj_hbm,
                               tmp, dvbig, m2, bi)
        return (m1, m2)

    m1, m2 = lax.fori_loop(0, nb_main >> 1, block2, (0, 0))

    m1 = _scan_extract(wli, cnt_i, wtv, ring1, sem1, owi_hbm,
                       tmp, dvbig, m1, TAIL_BI)
    m2 = _scan_extract(wlj, cnt_j, wttv, ring2, sem2, owj_hbm,
                       tmp, dvbig, m2, TAIL_BI)

    def drain_rest(m, semw, o_ref):
        def one(e, _):
            pltpu.make_async_copy(o_ref.at[pl.ds(0, DIM)], dv, semw).wait()
            return 0
        lax.fori_loop(0, m & 15, one, 0)

    drain_rest(m1, sem1, owi_hbm)
    drain_rest(m2, sem2, owj_hbm)


_k1 = functools.partial(
    pl.kernel,
    out_type=(
        jax.ShapeDtypeStruct((BATCH * DIM,), jnp.float32),
        jax.ShapeDtypeStruct((BATCH * DIM,), jnp.float32),
    ),
    mesh=plsc.VectorSubcoreMesh(core_axis_name="c", subcore_axis_name="s"),
    compiler_params=pltpu.CompilerParams(
        needs_layout_passes=False, use_tc_tiling_on_sc=True
    ),
    scratch_types=[
        pltpu.VMEM((4096,), jnp.int32),
        pltpu.VMEM((WLCAP,), jnp.int32),
        pltpu.VMEM((WLCAP,), jnp.int32),
        pltpu.VMEM((DIM, SLAB_W), jnp.float32),
        pltpu.VMEM((DIM, SLAB_W), jnp.float32),
        pltpu.VMEM((DIM, SLAB_W), jnp.float32),
        pltpu.VMEM((DIM, SLAB_W), jnp.float32),
        pltpu.VMEM((DIM, 128), jnp.float32),
        pltpu.VMEM((DIM, 128), jnp.float32),
        pltpu.VMEM((16, DIM), jnp.float32),
        pltpu.VMEM((16, DIM), jnp.float32),
        pltpu.VMEM((2 * L,), jnp.int32),
        pltpu.VMEM((16 * DIM,), jnp.float32),
        pltpu.VMEM((DIM,), jnp.float32),
        pltpu.SemaphoreType.DMA,
        pltpu.SemaphoreType.DMA,
        pltpu.SemaphoreType.DMA,
        pltpu.SemaphoreType.DMA,
    ],
)(_k1_body)


def _k2_body(wif_hbm, wjf_hbm, i_hbm, j_hbm, b_hbm, bt_hbm, out_hbm,
             iv, jv, biv, bjv, wiv, wjv, pv, sem):
    wid = lax.axis_index("s") * NC + lax.axis_index("c")
    base = wid * BPW

    pltpu.sync_copy(i_hbm.at[pl.ds(base, BPW)], iv.at[pl.ds(0, BPW)])
    pltpu.sync_copy(j_hbm.at[pl.ds(base, BPW)], jv.at[pl.ds(0, BPW)])
    zeros16i = jnp.zeros((L,), jnp.int32)
    iv[pl.ds(BPW, L)] = zeros16i
    jv[pl.ds(BPW, L)] = zeros16i

    cb = pltpu.async_copy(b_hbm.at[iv], biv, sem)
    cb.wait()
    cbt = pltpu.async_copy(bt_hbm.at[jv], bjv, sem)
    cbt.wait()

    pltpu.sync_copy(wif_hbm.at[pl.ds(base * DIM, BPW * DIM)], wiv)
    pltpu.sync_copy(wjf_hbm.at[pl.ds(base * DIM, BPW * DIM)], wjv)

    lane = lax.iota(jnp.int32, L)

    def pair(p, _):
        acc = jnp.zeros((L,), jnp.float32)
        for r in range(DIM // L):
            a = wiv[pl.ds(p * DIM + r * L, L)]
            c = wjv[pl.ds(p * DIM + r * L, L)]
            acc = acc + a * c
        bsum = biv[pl.ds(p, L)][0] + bjv[pl.ds(p, L)][0]
        acc = jnp.where(lane == 0, acc + bsum, acc)
        pv[p >> 3, pl.ds((p & 7) * L, L)] = acc
        return 0

    lax.fori_loop(0, BPW, pair, 0)
    pltpu.sync_copy(pv, out_hbm.at[pl.ds(wid * (BPW // 8), BPW // 8)])


_k2 = functools.partial(
    pl.kernel,
    out_type=jax.ShapeDtypeStruct((BATCH // 8, 128), jnp.float32),
    mesh=plsc.VectorSubcoreMesh(core_axis_name="c", subcore_axis_name="s"),
    compiler_params=pltpu.CompilerParams(
        needs_layout_passes=False, use_tc_tiling_on_sc=False
    ),
    scratch_types=[
        pltpu.VMEM((BPW + L,), jnp.int32),
        pltpu.VMEM((BPW + L,), jnp.int32),
        pltpu.VMEM((BPW + L,), jnp.float32),
        pltpu.VMEM((BPW + L,), jnp.float32),
        pltpu.VMEM((BPW * DIM,), jnp.float32),
        pltpu.VMEM((BPW * DIM,), jnp.float32),
        pltpu.VMEM((BPW // 8, 128), jnp.float32),
        pltpu.SemaphoreType.DMA,
    ],
)(_k2_body)


def _tc_loss_body(p_ref, x_ref, o_ref):
    p = p_ref[...]
    xr = x_ref[...]
    cc = lax.broadcasted_iota(jnp.int32, (128, 8), 0)
    mm = lax.broadcasted_iota(jnp.int32, (128, 8), 1)
    blk = jnp.where(cc // 16 == mm, 1.0, 0.0).astype(jnp.float32)
    s = jax.lax.dot_general(
        p, blk, (((1,), (0,)), ((), ())),
        precision=jax.lax.Precision.HIGHEST,
        preferred_element_type=jnp.float32,
    )
    diff = s - jnp.log(xr)
    w = jnp.clip((xr * (1.0 / X_MAX)) ** ALPHA, 0.0, 1.0)
    o_ref[0, 0] = jnp.sum(w * diff * diff) * (1.0 / BATCH)


_tc_loss = pl.pallas_call(
    _tc_loss_body,
    out_shape=jax.ShapeDtypeStruct((1, 1), jnp.float32),
    out_specs=pl.BlockSpec(memory_space=pltpu.MemorySpace.SMEM),
)


def kernel(i, j, x, W, W_tilde, b, b_tilde):
    i = i.astype(jnp.int32)
    j = j.astype(jnp.int32)
    wtail = jnp.pad(W[MAIN_END:].T, ((0, 0), (0, 128 - (VOCAB - MAIN_END))))
    wttail = jnp.pad(W_tilde[MAIN_END:].T,
                     ((0, 0), (0, 128 - (VOCAB - MAIN_END))))
    wi_flat, wj_flat = _k1(W.T, W_tilde.T, i, j, wtail, wttail)
    partials = _k2(wi_flat, wj_flat, i, j, b, b_tilde)
    out = _tc_loss(partials, x.reshape(BATCH // 8, 8))
    return out[0, 0]


# speedup vs baseline: 1.7736x; 1.1279x over previous
import functools

import jax
import jax.numpy as jnp
from jax import lax
from jax.experimental import pallas as pl
from jax.experimental.pallas import tpu as pltpu
from jax.experimental.pallas import tpu_sc as plsc

VOCAB = 1000000
DIM = 64
BATCH = 16384
X_MAX = 100.0
ALPHA = 0.75

NC = 2
NS = 16
NW = NC * NS
L = 16
BPW = BATCH // NW

RANGE = 31232
MAIN_END = 999936
SLAB_W = 512
NB = 62
TAIL_BI = NB
WLCAP = BATCH + L


def _scan_extract(wl, cnt, src, ring, semw, o_ref, tmp, dvbig, mtot, bi):
    lanes = lax.iota(jnp.int32, L)
    nvregs = (cnt + L - 1) >> 4

    def scanv(v, mtot):
        vals = wl[pl.ds(v * L, L)]
        mm = ((vals & 32767) >> 9) == bi
        mm = mm & ((lanes + v * L) < cnt)
        plsc.store_compressed(tmp.at[pl.ds(0, L)], vals, mask=mm)
        nv = plsc.all_reduce_population_count(mm)[0]

        def handle(e, mtot):
            packed = tmp[pl.ds(e, L)][0]
            kpair = packed >> 15
            cloc = (packed & 32767) - bi * SLAB_W
            colv = jnp.full((L,), 0, jnp.int32) + cloc
            slot = mtot & 15
            for r in range(DIM // L):
                ring[slot, pl.ds(r * L, L)] = plsc.load_gather(
                    src, [lanes + r * L, colv])
            pltpu.async_copy(
                ring.at[slot], o_ref.at[pl.ds(kpair * DIM, DIM)], semw)

            @pl.when(slot == 15)
            def _():
                pltpu.make_async_copy(
                    o_ref.at[pl.ds(0, 16 * DIM)], dvbig, semw).wait()

            return mtot + 1

        return lax.fori_loop(0, nv, handle, mtot)

    return lax.fori_loop(0, nvregs, scanv, mtot)


def _k1_body(pt_hbm, ptt_hbm, i_hbm, j_hbm, wtl_hbm, wttl_hbm,
             owi_hbm, owj_hbm,
             ivall, wli, wlj,
             bw0, bw1, wtv,
             ring1, ring2, tmp, dvbig, dv,
             semb0, semb1, sem1, sem2):
    wid = lax.axis_index("s") * NC + lax.axis_index("c")
    lo = wid * RANGE
    is_last = wid == NW - 1
    lanes = lax.iota(jnp.int32, L)

    ICH = 4096

    def build_wl(src_hbm, wl):
        hi = jnp.where(is_last, VOCAB, lo + RANGE)

        def chunk(ci, cnt):
            pltpu.sync_copy(src_hbm.at[pl.ds(ci * ICH, ICH)], ivall)

            def scan(v, cnt):
                vals = ivall[pl.ds(v * L, L)]
                m = (vals >= lo) & (vals < hi)
                packed = ((lanes + ci * ICH + v * L) << 15) | (vals - lo)
                plsc.store_compressed(wl.at[pl.ds(cnt, L)], packed, mask=m)
                return cnt + plsc.all_reduce_population_count(m)[0]

            return lax.fori_loop(0, ICH // L, scan, cnt)

        return lax.fori_loop(0, BATCH // ICH, chunk, 0)

    cnt_i = build_wl(i_hbm, wli)
    cnt_j = build_wl(j_hbm, wlj)

    bufs = ((bw0, semb0), (bw1, semb1))

    def stream_table(tbl_hbm, tl_hbm, wl, cnt, ring, semw, o_ref):
        pltpu.sync_copy(tl_hbm, wtv)

        def fire(bi, bw, semb):
            col = lo + bi * SLAB_W
            pltpu.async_copy(tbl_hbm.at[:, pl.ds(col, SLAB_W)], bw, semb)

        def drain_blk(bw, semb):
            pltpu.make_async_copy(
                tbl_hbm.at[:, pl.ds(0, SLAB_W)], bw, semb).wait()

        fire(0, *bufs[0])

        def block2(bi2, m):
            for b in (0, 1):
                bi = bi2 * 2 + b
                bw, semb = bufs[b]
                nbw, nsemb = bufs[1 - b]
                drain_blk(bw, semb)

                @pl.when(bi + 1 < NB)
                def _():
                    fire(bi + 1, nbw, nsemb)

                m = _scan_extract(wl, cnt, bw, ring, semw, o_ref,
                                  tmp, dvbig, m, bi)
            return m

        m = lax.fori_loop(0, NB // 2, block2, 0)
        m = _scan_extract(wl, cnt, wtv, ring, semw, o_ref,
                          tmp, dvbig, m, TAIL_BI)

        def one(e, _):
            pltpu.make_async_copy(o_ref.at[pl.ds(0, DIM)], dv, semw).wait()
            return 0

        lax.fori_loop(0, m & 15, one, 0)

    stream_table(pt_hbm, wtl_hbm, wli, cnt_i, ring1, sem1, owi_hbm)
    stream_table(ptt_hbm, wttl_hbm, wlj, cnt_j, ring2, sem2, owj_hbm)


_k1 = functools.partial(
    pl.kernel,
    out_type=(
        jax.ShapeDtypeStruct((BATCH * DIM,), jnp.float32),
        jax.ShapeDtypeStruct((BATCH * DIM,), jnp.float32),
    ),
    mesh=plsc.VectorSubcoreMesh(core_axis_name="c", subcore_axis_name="s"),
    compiler_params=pltpu.CompilerParams(
        needs_layout_passes=False, use_tc_tiling_on_sc=True
    ),
    scratch_types=[
        pltpu.VMEM((4096,), jnp.int32),
        pltpu.VMEM((WLCAP,), jnp.int32),
        pltpu.VMEM((WLCAP,), jnp.int32),
        pltpu.VMEM((DIM, SLAB_W), jnp.float32),
        pltpu.VMEM((DIM, SLAB_W), jnp.float32),
        pltpu.VMEM((DIM, 128), jnp.float32),
        pltpu.VMEM((16, DIM), jnp.float32),
        pltpu.VMEM((16, DIM), jnp.float32),
        pltpu.VMEM((2 * L,), jnp.int32),
        pltpu.VMEM((16 * DIM,), jnp.float32),
        pltpu.VMEM((DIM,), jnp.float32),
        pltpu.SemaphoreType.DMA,
        pltpu.SemaphoreType.DMA,
        pltpu.SemaphoreType.DMA,
        pltpu.SemaphoreType.DMA,
    ],
)(_k1_body)


def _k2_body(wif_hbm, wjf_hbm, i_hbm, j_hbm, b_hbm, bt_hbm, out_hbm,
             iv, jv, biv, bjv, wiv, wjv, pv, sem):
    wid = lax.axis_index("s") * NC + lax.axis_index("c")
    base = wid * BPW

    pltpu.sync_copy(i_hbm.at[pl.ds(base, BPW)], iv.at[pl.ds(0, BPW)])
    pltpu.sync_copy(j_hbm.at[pl.ds(base, BPW)], jv.at[pl.ds(0, BPW)])
    zeros16i = jnp.zeros((L,), jnp.int32)
    iv[pl.ds(BPW, L)] = zeros16i
    jv[pl.ds(BPW, L)] = zeros16i

    cb = pltpu.async_copy(b_hbm.at[iv], biv, sem)
    cb.wait()
    cbt = pltpu.async_copy(bt_hbm.at[jv], bjv, sem)
    cbt.wait()

    pltpu.sync_copy(wif_hbm.at[pl.ds(base * DIM, BPW * DIM)], wiv)
    pltpu.sync_copy(wjf_hbm.at[pl.ds(base * DIM, BPW * DIM)], wjv)

    lane = lax.iota(jnp.int32, L)

    def pair(p, _):
        acc = jnp.zeros((L,), jnp.float32)
        for r in range(DIM // L):
            a = wiv[pl.ds(p * DIM + r * L, L)]
            c = wjv[pl.ds(p * DIM + r * L, L)]
            acc = acc + a * c
        bsum = biv[pl.ds(p, L)][0] + bjv[pl.ds(p, L)][0]
        acc = jnp.where(lane == 0, acc + bsum, acc)
        pv[p >> 3, pl.ds((p & 7) * L, L)] = acc
        return 0

    lax.fori_loop(0, BPW, pair, 0)
    pltpu.sync_copy(pv, out_hbm.at[pl.ds(wid * (BPW // 8), BPW // 8)])


_k2 = functools.partial(
    pl.kernel,
    out_type=jax.ShapeDtypeStruct((BATCH // 8, 128), jnp.float32),
    mesh=plsc.VectorSubcoreMesh(core_axis_name="c", subcore_axis_name="s"),
    compiler_params=pltpu.CompilerParams(
        needs_layout_passes=False, use_tc_tiling_on_sc=False
    ),
    scratch_types=[
        pltpu.VMEM((BPW + L,), jnp.int32),
        pltpu.VMEM((BPW + L,), jnp.int32),
        pltpu.VMEM((BPW + L,), jnp.float32),
        pltpu.VMEM((BPW + L,), jnp.float32),
        pltpu.VMEM((BPW * DIM,), jnp.float32),
        pltpu.VMEM((BPW * DIM,), jnp.float32),
        pltpu.VMEM((BPW // 8, 128), jnp.float32),
        pltpu.SemaphoreType.DMA,
    ],
)(_k2_body)


def _tc_loss_body(p_ref, x_ref, o_ref):
    p = p_ref[...]
    xr = x_ref[...]
    cc = lax.broadcasted_iota(jnp.int32, (128, 8), 0)
    mm = lax.broadcasted_iota(jnp.int32, (128, 8), 1)
    blk = jnp.where(cc // 16 == mm, 1.0, 0.0).astype(jnp.float32)
    s = jax.lax.dot_general(
        p, blk, (((1,), (0,)), ((), ())),
        precision=jax.lax.Precision.HIGHEST,
        preferred_element_type=jnp.float32,
    )
    diff = s - jnp.log(xr)
    w = jnp.clip((xr * (1.0 / X_MAX)) ** ALPHA, 0.0, 1.0)
    o_ref[0, 0] = jnp.sum(w * diff * diff) * (1.0 / BATCH)


_tc_loss = pl.pallas_call(
    _tc_loss_body,
    out_shape=jax.ShapeDtypeStruct((1, 1), jnp.float32),
    out_specs=pl.BlockSpec(memory_space=pltpu.MemorySpace.SMEM),
)


def kernel(i, j, x, W, W_tilde, b, b_tilde):
    i = i.astype(jnp.int32)
    j = j.astype(jnp.int32)
    wtail = jnp.pad(W[MAIN_END:].T, ((0, 0), (0, 128 - (VOCAB - MAIN_END))))
    wttail = jnp.pad(W_tilde[MAIN_END:].T,
                     ((0, 0), (0, 128 - (VOCAB - MAIN_END))))
    wi_flat, wj_flat = _k1(W.T, W_tilde.T, i, j, wtail, wttail)
    partials = _k2(wi_flat, wj_flat, i, j, b, b_tilde)
    out = _tc_loss(partials, x.reshape(BATCH // 8, 8))
    return out[0, 0]
